# Initial kernel scaffold; baseline (speedup 1.0000x reference)
import jax
import jax.numpy as jnp
from jax.experimental import pallas as pl


def kernel(x, edge_index, edge_attr, batch, params):
    raise NotImplementedError("write your pallas kernel here")


# speedup vs baseline: 6.5864x; 6.5864x over previous
import functools
import math

import jax
import jax.numpy as jnp
from jax import lax
from jax.experimental import pallas as pl
from jax.experimental.pallas import tpu as pltpu
from jax.experimental.pallas import tpu_sc as plsc

NH = 16
TH = 128
FH = 64
HEADS = 8
SEEDS1 = 75
NGRAPHS = 400

NODES = 10000
NODES_PAD = 10240
EDGES = 160000
NW = 32
CL = 128
ECHUNKS = 40
EPT = ECHUNKS * CL
EPAD = NW * EPT
DUMMY = NODES

GPB = 8
NQ = GPB * SEEDS1
SCALE = 1.0 / math.sqrt(float(TH))
NEG = -1e30


def _mesh():
    return plsc.VectorSubcoreMesh(core_axis_name="c", subcore_axis_name="s")


@functools.lru_cache(maxsize=None)
def _make_sc_gather(D):
    @functools.partial(
        pl.kernel, mesh=_mesh(),
        compiler_params=pltpu.CompilerParams(use_tc_tiling_on_sc=False),
        out_type=jax.ShapeDtypeStruct((EPAD, D), jnp.float32),
        scratch_types=[
            pltpu.VMEM((ECHUNKS, CL), jnp.int32),
            pltpu.VMEM((CL, D), jnp.float32),
            pltpu.SemaphoreType.DMA,
        ],
    )
    def gather(table, idx, out, idx_v, buf, sem):
        c = lax.axis_index("c")
        s = lax.axis_index("s")
        wid = s * 2 + c
        pltpu.sync_copy(idx.at[wid], idx_v)

        def body(j, carry):
            pltpu.async_copy(table.at[idx_v.at[j]], buf, sem).wait()
            pltpu.sync_copy(buf, out.at[pl.ds(wid * EPT + j * CL, CL)])
            return carry

        lax.fori_loop(0, ECHUNKS, body, 0)

    return gather


@functools.lru_cache(maxsize=None)
def _make_sc_scatter(D):
    @functools.partial(
        pl.kernel, mesh=_mesh(),
        compiler_params=pltpu.CompilerParams(use_tc_tiling_on_sc=False),
        out_type=jax.ShapeDtypeStruct((2, NODES_PAD, D), jnp.float32),
        scratch_types=[
            pltpu.VMEM((ECHUNKS, CL), jnp.int32),
            pltpu.VMEM((CL, D), jnp.float32),
            pltpu.VMEM_SHARED((NODES_PAD, D), jnp.float32),
        ],
    )
    def scatter(msg, idx, zero, out, idx_v, buf, shared):
        c = lax.axis_index("c")
        s = lax.axis_index("s")
        wid = s * 2 + c

        @pl.when(s == 0)
        def _():
            pltpu.sync_copy(zero, shared)

        plsc.subcore_barrier()
        pltpu.sync_copy(idx.at[wid], idx_v)

        def body(j, carry):
            pltpu.sync_copy(msg.at[pl.ds(wid * EPT + j * CL, CL)], buf)
            pltpu.sync_copy(buf, shared.at[idx_v.at[j]], add=True)
            return carry

        lax.fori_loop(0, ECHUNKS, body, 0)
        plsc.subcore_barrier()

        @pl.when(s == 0)
        def _():
            pltpu.sync_copy(shared, out.at[c])

    return scatter


@functools.lru_cache(maxsize=None)
def _make_sc_gcn(D):
    @functools.partial(
        pl.kernel, mesh=_mesh(),
        compiler_params=pltpu.CompilerParams(use_tc_tiling_on_sc=False),
        out_type=jax.ShapeDtypeStruct((2, NODES_PAD, D), jnp.float32),
        scratch_types=[
            pltpu.VMEM((ECHUNKS, CL), jnp.int32),
            pltpu.VMEM((ECHUNKS, CL), jnp.int32),
            pltpu.VMEM((CL, D), jnp.float32),
            pltpu.VMEM_SHARED((NODES_PAD, D), jnp.float32),
            pltpu.SemaphoreType.DMA,
        ],
    )
    def gcn(table, sidx, didx, zero, out, si_v, di_v, buf, shared, sem):
        c = lax.axis_index("c")
        s = lax.axis_index("s")
        wid = s * 2 + c

        @pl.when(s == 0)
        def _():
            pltpu.sync_copy(zero, shared)

        plsc.subcore_barrier()
        pltpu.sync_copy(sidx.at[wid], si_v)
        pltpu.sync_copy(didx.at[wid], di_v)

        def body(j, carry):
            pltpu.async_copy(table.at[si_v.at[j]], buf, sem).wait()
            pltpu.sync_copy(buf, shared.at[di_v.at[j]], add=True)
            return carry

        lax.fori_loop(0, ECHUNKS, body, 0)
        plsc.subcore_barrier()

        @pl.when(s == 0)
        def _():
            pltpu.sync_copy(shared, out.at[c])

    return gcn


def _sc_gather16(table, idx):
    return _make_sc_gather(NH)(table, idx)


def _sc_scatter16(msg, idx, zero):
    return _make_sc_scatter(NH)(msg, idx, zero)


def _sc_gcn128(table, sidx, didx, zero):
    return _make_sc_gcn(TH)(table, sidx, didx, zero)


def _dot(a, b):
    return jnp.dot(a, b, precision=lax.Precision.HIGHEST,
                   preferred_element_type=jnp.float32)


def _dotg(a, b, dims):
    return lax.dot_general(a, b, (dims, ((), ())),
                           precision=lax.Precision.HIGHEST,
                           preferred_element_type=jnp.float32)


def _node_proj(x_pad, W, b):
    def body(x_ref, w_ref, b_ref, o_ref):
        o_ref[...] = jnp.maximum(_dot(x_ref[...], w_ref[...]) + b_ref[...], 0.0)

    return pl.pallas_call(
        body,
        grid=(NODES_PAD // 1024,),
        in_specs=[
            pl.BlockSpec((1024, 37), lambda i: (i, 0)),
            pl.BlockSpec((37, NH), lambda i: (0, 0)),
            pl.BlockSpec((1, NH), lambda i: (0, 0)),
        ],
        out_specs=pl.BlockSpec((1024, NH), lambda i: (i, 0)),
        out_shape=jax.ShapeDtypeStruct((NODES_PAD, NH), jnp.float32),
    )(x_pad, W, b)


def _edge_mlp(ea_pad, W1, b1, W2, b2):
    def body(e_ref, w1_ref, b1_ref, w2_ref, b2_ref, o_ref):
        e = jnp.maximum(_dot(e_ref[...], w1_ref[...]) + b1_ref[...], 0.0)
        o_ref[...] = _dot(e, w2_ref[...]) + b2_ref[...]

    return pl.pallas_call(
        body,
        grid=(EPAD // 2048,),
        in_specs=[
            pl.BlockSpec((2048, 8), lambda i: (i, 0)),
            pl.BlockSpec((8, 128), lambda i: (0, 0)),
            pl.BlockSpec((1, 128), lambda i: (0, 0)),
            pl.BlockSpec((128, 256), lambda i: (0, 0)),
            pl.BlockSpec((1, 256), lambda i: (0, 0)),
        ],
        out_specs=pl.BlockSpec((2048, 256), lambda i: (i, 0)),
        out_shape=jax.ShapeDtypeStruct((EPAD, 256), jnp.float32),
    )(ea_pad, W1, b1, W2, b2)


def _tc_msg(xs, theta):
    def body(xs_ref, th_ref, o_ref):
        xsv = xs_ref[...]
        th = th_ref[...]
        acc = xsv[:, 0:1] * th[:, 0:NH]
        for h in range(1, NH):
            acc = acc + xsv[:, h:h + 1] * th[:, NH * h:NH * h + NH]
        o_ref[...] = acc

    return pl.pallas_call(
        body,
        grid=(EPAD // 2048,),
        in_specs=[
            pl.BlockSpec((2048, NH), lambda i: (i, 0)),
            pl.BlockSpec((2048, 256), lambda i: (i, 0)),
        ],
        out_specs=pl.BlockSpec((2048, NH), lambda i: (i, 0)),
        out_shape=jax.ShapeDtypeStruct((EPAD, NH), jnp.float32),
    )(xs, theta)


def _tc_update(p0, p1, node, Wroot, broot, WihT, bih, WhhT, bhh):
    def body(p0_ref, p1_ref, n_ref, wr_ref, br_ref, wi_ref, bi_ref,
             wh_ref, bh_ref, o_ref):
        node_v = n_ref[...]
        agg = p0_ref[...] + p1_ref[...]
        a = jnp.maximum(agg + _dot(node_v, wr_ref[...]) + br_ref[...], 0.0)
        gi = _dot(a, wi_ref[...]) + bi_ref[...]
        gh = _dot(node_v, wh_ref[...]) + bh_ref[...]
        r = jax.nn.sigmoid(gi[:, :NH] + gh[:, :NH])
        z = jax.nn.sigmoid(gi[:, NH:2 * NH] + gh[:, NH:2 * NH])
        n = jnp.tanh(gi[:, 2 * NH:] + r * gh[:, 2 * NH:])
        o_ref[...] = (1.0 - z) * n + z * node_v

    return pl.pallas_call(
        body,
        grid=(NODES_PAD // 1024,),
        in_specs=[
            pl.BlockSpec((1024, NH), lambda i: (i, 0)),
            pl.BlockSpec((1024, NH), lambda i: (i, 0)),
            pl.BlockSpec((1024, NH), lambda i: (i, 0)),
            pl.BlockSpec((NH, NH), lambda i: (0, 0)),
            pl.BlockSpec((1, NH), lambda i: (0, 0)),
            pl.BlockSpec((NH, 3 * NH), lambda i: (0, 0)),
            pl.BlockSpec((1, 3 * NH), lambda i: (0, 0)),
            pl.BlockSpec((NH, 3 * NH), lambda i: (0, 0)),
            pl.BlockSpec((1, 3 * NH), lambda i: (0, 0)),
        ],
        out_specs=pl.BlockSpec((1024, NH), lambda i: (i, 0)),
        out_shape=jax.ShapeDtypeStruct((NODES_PAD, NH), jnp.float32),
    )(p0, p1, node, Wroot, broot, WihT, bih, WhhT, bhh)


def _tc_pre_gcn(node, d0, d1, lin1W, lin1b, Wk, Wv):
    def body(n_ref, d0_ref, d1_ref, l1_ref, b1_ref, wk_ref, wv_ref,
             hk_ref, hv_ref, di_ref):
        deg = d0_ref[:, 0:1] + d1_ref[:, 0:1] + 1.0
        dinv = lax.rsqrt(deg)
        xL = _dot(n_ref[...], l1_ref[...]) + b1_ref[...]
        hk_ref[...] = dinv * _dot(xL, wk_ref[...])
        hv_ref[...] = dinv * _dot(xL, wv_ref[...])
        di_ref[...] = dinv

    return pl.pallas_call(
        body,
        grid=(NODES_PAD // 1024,),
        in_specs=[
            pl.BlockSpec((1024, NH), lambda i: (i, 0)),
            pl.BlockSpec((1024, NH), lambda i: (i, 0)),
            pl.BlockSpec((1024, NH), lambda i: (i, 0)),
            pl.BlockSpec((NH, TH), lambda i: (0, 0)),
            pl.BlockSpec((1, TH), lambda i: (0, 0)),
            pl.BlockSpec((TH, TH), lambda i: (0, 0)),
            pl.BlockSpec((TH, TH), lambda i: (0, 0)),
        ],
        out_specs=[
            pl.BlockSpec((1024, TH), lambda i: (i, 0)),
            pl.BlockSpec((1024, TH), lambda i: (i, 0)),
            pl.BlockSpec((1024, 1), lambda i: (i, 0)),
        ],
        out_shape=[
            jax.ShapeDtypeStruct((NODES_PAD, TH), jnp.float32),
            jax.ShapeDtypeStruct((NODES_PAD, TH), jnp.float32),
            jax.ShapeDtypeStruct((NODES_PAD, 1), jnp.float32),
        ],
    )(node, d0, d1, lin1W, lin1b, Wk, Wv)


def _tc_post_gcn(pK0, pK1, hsK, pV0, pV1, hsV, dinv, bk, bv):
    def body(k0, k1, hk, v0, v1, hv, di, bk_ref, bv_ref, ko_ref, vo_ref):
        d = di[...]
        ko_ref[...] = d * (k0[...] + k1[...] + hk[...]) + bk_ref[...]
        vo_ref[...] = d * (v0[...] + v1[...] + hv[...]) + bv_ref[...]

    spec = pl.BlockSpec((1024, TH), lambda i: (i, 0))
    return pl.pallas_call(
        body,
        grid=(NODES_PAD // 1024,),
        in_specs=[spec, spec, spec, spec, spec, spec,
                  pl.BlockSpec((1024, 1), lambda i: (i, 0)),
                  pl.BlockSpec((1, TH), lambda i: (0, 0)),
                  pl.BlockSpec((1, TH), lambda i: (0, 0))],
        out_specs=[spec, spec],
        out_shape=[
            jax.ShapeDtypeStruct((NODES_PAD, TH), jnp.float32),
            jax.ShapeDtypeStruct((NODES_PAD, TH), jnp.float32),
        ],
    )(pK0, pK1, hsK, pV0, pV1, hsV, dinv, bk, bv)


def _heads(t):
    return [t[:, h * NH:(h + 1) * NH] for h in range(HEADS)]


def _attend_tail(o, Wo, bo):
    return o + jnp.maximum(_dot(o, Wo) + bo, 0.0)


def _tc_attn(starts, Kd, Vd, Qs, q3s, w):
    def body(starts_ref, kd_ref, vd_ref, qs_ref, q3_ref,
             wo1, bo1, wq, bq, wk, bk, wv, bv, wso, bso,
             wk2, bk2, wv2, bv2, wo2, bo2,
             l2w, l2b, f0w, f0b, fw, fb, o_ref):
        g = pl.program_id(0)
        base = g * GPB
        s_g = [starts_ref[base + i] for i in range(GPB + 1)]
        start0 = s_g[0]
        nch = (s_g[GPB] - start0 + CL - 1) // CL

        rg = lax.broadcasted_iota(jnp.int32, (NQ, 1), 0) // SEEDS1
        rs = jnp.zeros((NQ, 1), jnp.int32)
        re = jnp.zeros((NQ, 1), jnp.int32)
        for gg in range(GPB):
            rs = jnp.where(rg == gg, s_g[gg], rs)
            re = jnp.where(rg == gg, s_g[gg + 1], re)

        Qsv = qs_ref[...]
        Qh = _heads(Qsv)

        def chunk(j, carry):
            off = start0 + j * CL
            kc = kd_ref[pl.ds(off, CL), :]
            vc = vd_ref[pl.ds(off, CL), :]
            pos = off + lax.broadcasted_iota(jnp.int32, (1, CL), 1)
            valid = (pos >= rs) & (pos < re)
            kch = _heads(kc)
            vch = _heads(vc)
            new = []
            for h in range(HEADS):
                m, l, acc = carry[h]
                sc = _dotg(Qh[h], kch[h], ((1,), (1,))) * SCALE
                sc = jnp.where(valid, sc, NEG)
                mnew = jnp.maximum(m, jnp.max(sc, axis=1, keepdims=True))
                alpha = jnp.exp(m - mnew)
                p = jnp.where(valid, jnp.exp(sc - mnew), 0.0)
                lnew = l * alpha + jnp.sum(p, axis=1, keepdims=True)
                accnew = acc * alpha + _dotg(p, vch[h], ((1,), (0,)))
                new.append((mnew, lnew, accnew))
            return tuple(new)

        init = tuple((jnp.full((NQ, 1), NEG, jnp.float32),
                      jnp.zeros((NQ, 1), jnp.float32),
                      jnp.zeros((NQ, NH), jnp.float32)) for _ in range(HEADS))
        fin = lax.fori_loop(0, nch, chunk, init)
        outs = []
        for h in range(HEADS):
            m, l, acc = fin[h]
            outs.append(Qh[h] + acc / jnp.maximum(l, 1e-20))
        o1 = jnp.concatenate(outs, axis=1)
        h1 = _attend_tail(o1, wo1[...], bo1[...])

        qs_ = _dot(h1, wq[...]) + bq[...]
        ks_ = _dot(h1, wk[...]) + bk[...]
        vs_ = _dot(h1, wv[...]) + bv[...]
        cg = lax.broadcasted_iota(jnp.int32, (1, NQ), 1) // SEEDS1
        bm = rg == cg
        qh2, kh2, vh2 = _heads(qs_), _heads(ks_), _heads(vs_)
        outs2 = []
        for h in range(HEADS):
            sc = _dotg(qh2[h], kh2[h], ((1,), (1,))) * SCALE
            sc = jnp.where(bm, sc, NEG)
            m = jnp.max(sc, axis=1, keepdims=True)
            p = jnp.exp(sc - m)
            p = jnp.where(bm, p, 0.0)
            A = p / jnp.sum(p, axis=1, keepdims=True)
            outs2.append(qh2[h] + _dotg(A, vh2[h], ((1,), (0,))))
        o2 = jnp.concatenate(outs2, axis=1)
        h2 = _attend_tail(o2, wso[...], bso[...])

        q3v = q3_ref[...]
        k3 = _dot(h2, wk2[...]) + bk2[...]
        v3 = _dot(h2, wv2[...]) + bv2[...]
        qg = lax.broadcasted_iota(jnp.int32, (GPB, 1), 0)
        qm = qg == cg
        q3h, k3h, v3h = _heads(q3v), _heads(k3), _heads(v3)
        outs3 = []
        for h in range(HEADS):
            sc = _dotg(q3h[h], k3h[h], ((1,), (1,))) * SCALE
            sc = jnp.where(qm, sc, NEG)
            m = jnp.max(sc, axis=1, keepdims=True)
            p = jnp.exp(sc - m)
            p = jnp.where(qm, p, 0.0)
            A = p / jnp.sum(p, axis=1, keepdims=True)
            outs3.append(q3h[h] + _dotg(A, v3h[h], ((1,), (0,))))
        o3 = jnp.concatenate(outs3, axis=1)
        h3 = _attend_tail(o3, wo2[...], bo2[...])

        g1 = _dot(h3, l2w[...]) + l2b[...]
        g2 = jnp.maximum(_dot(g1, f0w[...]) + f0b[...], 0.0)
        o_ref[...] = _dot(g2, fw[...]) + fb[...]

    full = lambda a: pl.BlockSpec(a.shape, lambda g, s: (0,) * a.ndim)
    win = [w['pma1_fco_W'], w['pma1_fco_b'],
           w['sab_fcq_W'], w['sab_fcq_b'], w['sab_fck_W'], w['sab_fck_b'],
           w['sab_fcv_W'], w['sab_fcv_b'], w['sab_fco_W'], w['sab_fco_b'],
           w['pma2_fck_W'], w['pma2_fck_b'], w['pma2_fcv_W'], w['pma2_fcv_b'],
           w['pma2_fco_W'], w['pma2_fco_b'],
           w['lin2_W'], w['lin2_b'], w['fc0_W'], w['fc0_b'],
           w['final_W'], w['final_b']]
    grid_spec = pltpu.PrefetchScalarGridSpec(
        num_scalar_prefetch=1,
        grid=(NGRAPHS // GPB,),
        in_specs=[full(Kd), full(Vd), full(Qs), full(q3s)] +
                 [full(a) for a in win],
        out_specs=pl.BlockSpec((GPB, 1), lambda g, s: (g, 0)),
    )
    return pl.pallas_call(
        body,
        grid_spec=grid_spec,
        out_shape=jax.ShapeDtypeStruct((NGRAPHS, 1), jnp.float32),
    )(starts, Kd, Vd, Qs, q3s, *win)


def kernel(x, edge_index, edge_attr, batch, params):
    p = params
    f32 = jnp.float32
    i32 = jnp.int32
    src = edge_index[0].astype(i32)
    dst = edge_index[1].astype(i32)

    epad = EPAD - EDGES
    src_idx = jnp.concatenate([src, jnp.zeros((epad,), i32)]).reshape(NW, ECHUNKS, CL)
    dst_idx = jnp.concatenate([dst, jnp.full((epad,), DUMMY, i32)]).reshape(NW, ECHUNKS, CL)
    x_pad = jnp.pad(x, ((0, NODES_PAD - NODES), (0, 0)))
    ea_pad = jnp.pad(edge_attr, ((0, epad), (0, 2)))
    zeros16 = jnp.zeros((NODES_PAD, NH), f32)
    zeros128 = jnp.zeros((NODES_PAD, TH), f32)
    ones_msg = jnp.ones((EPAD, NH), f32)

    r1 = lambda a: a.reshape(1, -1)
    W1p = jnp.pad(p['We1'], ((0, 2), (0, 0)))

    node = _node_proj(x_pad, p['proj_W'], r1(p['proj_b']))
    theta = _edge_mlp(ea_pad, W1p, r1(p['be1']), p['We2'], r1(p['be2']))
    deg_parts = _sc_scatter16(ones_msg, dst_idx, zeros16)

    WihT = p['W_ih'].T
    WhhT = p['W_hh'].T
    h = node
    for _ in range(3):
        xs = _sc_gather16(h, src_idx)
        msg = _tc_msg(xs, theta)
        parts = _sc_scatter16(msg, dst_idx, zeros16)
        h = _tc_update(parts[0], parts[1], h, p['Wroot'], r1(p['broot']),
                       WihT, r1(p['b_ih']), WhhT, r1(p['b_hh']))

    hsK, hsV, dinv = _tc_pre_gcn(h, deg_parts[0], deg_parts[1],
                                 p['lin1_W'], r1(p['lin1_b']),
                                 p['pma1_k_W'], p['pma1_v_W'])
    pK = _sc_gcn128(hsK, src_idx, dst_idx, zeros128)
    pV = _sc_gcn128(hsV, src_idx, dst_idx, zeros128)
    Kd, Vd = _tc_post_gcn(pK[0], pK[1], hsK, pV[0], pV[1], hsV, dinv,
                          r1(p['pma1_k_b']), r1(p['pma1_v_b']))

    Qp = p['S1'][0] @ p['pma1_fcq_W'] + p['pma1_fcq_b']
    Qs = jnp.tile(Qp, (GPB, 1))
    q3 = p['S2'][0] @ p['pma2_fcq_W'] + p['pma2_fcq_b']
    q3s = jnp.tile(q3, (GPB, 1))

    starts = jnp.searchsorted(batch.astype(i32), jnp.arange(NGRAPHS + 1, dtype=i32)).astype(i32)

    wdict = {k: (p[k] if k.endswith('_W') else r1(p[k]))
             for k in ['pma1_fco_W', 'pma1_fco_b',
                       'sab_fcq_W', 'sab_fcq_b', 'sab_fck_W', 'sab_fck_b',
                       'sab_fcv_W', 'sab_fcv_b', 'sab_fco_W', 'sab_fco_b',
                       'pma2_fck_W', 'pma2_fck_b', 'pma2_fcv_W', 'pma2_fcv_b',
                       'pma2_fco_W', 'pma2_fco_b',
                       'lin2_W', 'lin2_b', 'fc0_W', 'fc0_b',
                       'final_W', 'final_b']}
    return _tc_attn(starts, Kd, Vd, Qs, q3s, wdict)


# speedup vs baseline: 6.9766x; 1.0592x over previous
import functools
import math

import jax
import jax.numpy as jnp
from jax import lax
from jax.experimental import pallas as pl
from jax.experimental.pallas import tpu as pltpu
from jax.experimental.pallas import tpu_sc as plsc

NH = 16
TH = 128
FH = 64
HEADS = 8
SEEDS1 = 75
NGRAPHS = 400

NODES = 10000
NODES_PAD = 10240
EDGES = 160000
NW = 32
CL = 128
ECHUNKS = 40
EPT = ECHUNKS * CL
EPAD = NW * EPT
DUMMY = NODES

GPB = 8
NQ = GPB * SEEDS1
SCALE = 1.0 / math.sqrt(float(TH))
NEG = -1e30


def _mesh():
    return plsc.VectorSubcoreMesh(core_axis_name="c", subcore_axis_name="s")


@functools.lru_cache(maxsize=None)
def _make_sc_gather(D):
    @functools.partial(
        pl.kernel, mesh=_mesh(),
        compiler_params=pltpu.CompilerParams(use_tc_tiling_on_sc=False),
        out_type=jax.ShapeDtypeStruct((EPAD, D), jnp.float32),
        scratch_types=[
            pltpu.VMEM((ECHUNKS, CL), jnp.int32),
            pltpu.VMEM((CL, D), jnp.float32),
            pltpu.SemaphoreType.DMA,
        ],
    )
    def gather(table, idx, out, idx_v, buf, sem):
        c = lax.axis_index("c")
        s = lax.axis_index("s")
        wid = s * 2 + c
        pltpu.sync_copy(idx.at[wid], idx_v)

        def body(j, carry):
            pltpu.async_copy(table.at[idx_v.at[j]], buf, sem).wait()
            pltpu.sync_copy(buf, out.at[pl.ds(wid * EPT + j * CL, CL)])
            return carry

        lax.fori_loop(0, ECHUNKS, body, 0)

    return gather


@functools.lru_cache(maxsize=None)
def _make_sc_scatter(D):
    @functools.partial(
        pl.kernel, mesh=_mesh(),
        compiler_params=pltpu.CompilerParams(use_tc_tiling_on_sc=False),
        out_type=jax.ShapeDtypeStruct((2, NODES_PAD, D), jnp.float32),
        scratch_types=[
            pltpu.VMEM((ECHUNKS, CL), jnp.int32),
            pltpu.VMEM((CL, D), jnp.float32),
            pltpu.VMEM_SHARED((NODES_PAD, D), jnp.float32),
        ],
    )
    def scatter(msg, idx, zero, out, idx_v, buf, shared):
        c = lax.axis_index("c")
        s = lax.axis_index("s")
        wid = s * 2 + c

        @pl.when(s == 0)
        def _():
            pltpu.sync_copy(zero, shared)

        plsc.subcore_barrier()
        pltpu.sync_copy(idx.at[wid], idx_v)

        def body(j, carry):
            pltpu.sync_copy(msg.at[pl.ds(wid * EPT + j * CL, CL)], buf)
            pltpu.sync_copy(buf, shared.at[idx_v.at[j]], add=True)
            return carry

        lax.fori_loop(0, ECHUNKS, body, 0)
        plsc.subcore_barrier()

        @pl.when(s == 0)
        def _():
            pltpu.sync_copy(shared, out.at[c])

    return scatter


@functools.lru_cache(maxsize=None)
def _make_sc_gcn(D):
    @functools.partial(
        pl.kernel, mesh=_mesh(),
        compiler_params=pltpu.CompilerParams(use_tc_tiling_on_sc=False),
        out_type=jax.ShapeDtypeStruct((2, NODES_PAD, D), jnp.float32),
        scratch_types=[
            pltpu.VMEM((ECHUNKS, CL), jnp.int32),
            pltpu.VMEM((ECHUNKS, CL), jnp.int32),
            pltpu.VMEM((CL, D), jnp.float32),
            pltpu.VMEM_SHARED((NODES_PAD, D), jnp.float32),
            pltpu.SemaphoreType.DMA,
        ],
    )
    def gcn(table, sidx, didx, zero, out, si_v, di_v, buf, shared, sem):
        c = lax.axis_index("c")
        s = lax.axis_index("s")
        wid = s * 2 + c

        @pl.when(s == 0)
        def _():
            pltpu.sync_copy(zero, shared)

        plsc.subcore_barrier()
        pltpu.sync_copy(sidx.at[wid], si_v)
        pltpu.sync_copy(didx.at[wid], di_v)

        def body(j, carry):
            pltpu.async_copy(table.at[si_v.at[j]], buf, sem).wait()
            pltpu.sync_copy(buf, shared.at[di_v.at[j]], add=True)
            return carry

        lax.fori_loop(0, ECHUNKS, body, 0)
        plsc.subcore_barrier()

        @pl.when(s == 0)
        def _():
            pltpu.sync_copy(shared, out.at[c])

    return gcn


def _sc_gather16(table, idx):
    return _make_sc_gather(NH)(table, idx)


def _sc_scatter16(msg, idx, zero):
    return _make_sc_scatter(NH)(msg, idx, zero)


def _sc_gcn128(table, sidx, didx, zero):
    return _make_sc_gcn(TH)(table, sidx, didx, zero)


def _dot(a, b):
    return jnp.dot(a, b, precision=lax.Precision.HIGHEST,
                   preferred_element_type=jnp.float32)


def _dotg(a, b, dims):
    return lax.dot_general(a, b, (dims, ((), ())),
                           precision=lax.Precision.HIGHEST,
                           preferred_element_type=jnp.float32)


def _doth(a, b):
    return jnp.dot(a, b, precision=lax.Precision.HIGHEST,
                   preferred_element_type=jnp.float32)


def _node_proj(x_pad, W, b):
    def body(x_ref, w_ref, b_ref, o_ref):
        o_ref[...] = jnp.maximum(_dot(x_ref[...], w_ref[...]) + b_ref[...], 0.0)

    return pl.pallas_call(
        body,
        grid=(NODES_PAD // 1024,),
        in_specs=[
            pl.BlockSpec((1024, 37), lambda i: (i, 0)),
            pl.BlockSpec((37, NH), lambda i: (0, 0)),
            pl.BlockSpec((1, NH), lambda i: (0, 0)),
        ],
        out_specs=pl.BlockSpec((1024, NH), lambda i: (i, 0)),
        out_shape=jax.ShapeDtypeStruct((NODES_PAD, NH), jnp.float32),
    )(x_pad, W, b)


def _edge_mlp(ea_pad, W1, b1, W2, b2):
    def body(e_ref, w1_ref, b1_ref, w2_ref, b2_ref, o_ref):
        e = jnp.maximum(_dot(e_ref[...], w1_ref[...]) + b1_ref[...], 0.0)
        o_ref[...] = _dot(e, w2_ref[...]) + b2_ref[...]

    return pl.pallas_call(
        body,
        grid=(EPAD // 2048,),
        in_specs=[
            pl.BlockSpec((2048, 8), lambda i: (i, 0)),
            pl.BlockSpec((8, 128), lambda i: (0, 0)),
            pl.BlockSpec((1, 128), lambda i: (0, 0)),
            pl.BlockSpec((128, 256), lambda i: (0, 0)),
            pl.BlockSpec((1, 256), lambda i: (0, 0)),
        ],
        out_specs=pl.BlockSpec((2048, 256), lambda i: (i, 0)),
        out_shape=jax.ShapeDtypeStruct((EPAD, 256), jnp.float32),
    )(ea_pad, W1, b1, W2, b2)


def _tc_msg(xs, theta):
    def body(xs_ref, th_ref, o_ref):
        xsv = xs_ref[...]
        th = th_ref[...]
        acc = xsv[:, 0:1] * th[:, 0:NH]
        for h in range(1, NH):
            acc = acc + xsv[:, h:h + 1] * th[:, NH * h:NH * h + NH]
        o_ref[...] = acc

    return pl.pallas_call(
        body,
        grid=(EPAD // 2048,),
        in_specs=[
            pl.BlockSpec((2048, NH), lambda i: (i, 0)),
            pl.BlockSpec((2048, 256), lambda i: (i, 0)),
        ],
        out_specs=pl.BlockSpec((2048, NH), lambda i: (i, 0)),
        out_shape=jax.ShapeDtypeStruct((EPAD, NH), jnp.float32),
    )(xs, theta)


def _tc_update(p0, p1, node, Wroot, broot, WihT, bih, WhhT, bhh):
    def body(p0_ref, p1_ref, n_ref, wr_ref, br_ref, wi_ref, bi_ref,
             wh_ref, bh_ref, o_ref):
        node_v = n_ref[...]
        agg = p0_ref[...] + p1_ref[...]
        a = jnp.maximum(agg + _dot(node_v, wr_ref[...]) + br_ref[...], 0.0)
        gi = _dot(a, wi_ref[...]) + bi_ref[...]
        gh = _dot(node_v, wh_ref[...]) + bh_ref[...]
        r = jax.nn.sigmoid(gi[:, :NH] + gh[:, :NH])
        z = jax.nn.sigmoid(gi[:, NH:2 * NH] + gh[:, NH:2 * NH])
        n = jnp.tanh(gi[:, 2 * NH:] + r * gh[:, 2 * NH:])
        o_ref[...] = (1.0 - z) * n + z * node_v

    return pl.pallas_call(
        body,
        grid=(NODES_PAD // 1024,),
        in_specs=[
            pl.BlockSpec((1024, NH), lambda i: (i, 0)),
            pl.BlockSpec((1024, NH), lambda i: (i, 0)),
            pl.BlockSpec((1024, NH), lambda i: (i, 0)),
            pl.BlockSpec((NH, NH), lambda i: (0, 0)),
            pl.BlockSpec((1, NH), lambda i: (0, 0)),
            pl.BlockSpec((NH, 3 * NH), lambda i: (0, 0)),
            pl.BlockSpec((1, 3 * NH), lambda i: (0, 0)),
            pl.BlockSpec((NH, 3 * NH), lambda i: (0, 0)),
            pl.BlockSpec((1, 3 * NH), lambda i: (0, 0)),
        ],
        out_specs=pl.BlockSpec((1024, NH), lambda i: (i, 0)),
        out_shape=jax.ShapeDtypeStruct((NODES_PAD, NH), jnp.float32),
    )(p0, p1, node, Wroot, broot, WihT, bih, WhhT, bhh)


def _tc_pre_gcn(node, d0, d1, lin1W, lin1b, Wk, Wv):
    def body(n_ref, d0_ref, d1_ref, l1_ref, b1_ref, wk_ref, wv_ref,
             hk_ref, hv_ref, di_ref):
        deg = d0_ref[:, 0:1] + d1_ref[:, 0:1] + 1.0
        dinv = lax.rsqrt(deg)
        xL = _dot(n_ref[...], l1_ref[...]) + b1_ref[...]
        hk_ref[...] = dinv * _dot(xL, wk_ref[...])
        hv_ref[...] = dinv * _dot(xL, wv_ref[...])
        di_ref[...] = dinv

    return pl.pallas_call(
        body,
        grid=(NODES_PAD // 1024,),
        in_specs=[
            pl.BlockSpec((1024, NH), lambda i: (i, 0)),
            pl.BlockSpec((1024, NH), lambda i: (i, 0)),
            pl.BlockSpec((1024, NH), lambda i: (i, 0)),
            pl.BlockSpec((NH, TH), lambda i: (0, 0)),
            pl.BlockSpec((1, TH), lambda i: (0, 0)),
            pl.BlockSpec((TH, TH), lambda i: (0, 0)),
            pl.BlockSpec((TH, TH), lambda i: (0, 0)),
        ],
        out_specs=[
            pl.BlockSpec((1024, TH), lambda i: (i, 0)),
            pl.BlockSpec((1024, TH), lambda i: (i, 0)),
            pl.BlockSpec((1024, 1), lambda i: (i, 0)),
        ],
        out_shape=[
            jax.ShapeDtypeStruct((NODES_PAD, TH), jnp.float32),
            jax.ShapeDtypeStruct((NODES_PAD, TH), jnp.float32),
            jax.ShapeDtypeStruct((NODES_PAD, 1), jnp.float32),
        ],
    )(node, d0, d1, lin1W, lin1b, Wk, Wv)


def _tc_post_gcn(pK0, pK1, hsK, pV0, pV1, hsV, dinv, bk, bv):
    def body(k0, k1, hk, v0, v1, hv, di, bk_ref, bv_ref, ko_ref, vo_ref):
        d = di[...]
        ko_ref[...] = d * (k0[...] + k1[...] + hk[...]) + bk_ref[...]
        vo_ref[...] = d * (v0[...] + v1[...] + hv[...]) + bv_ref[...]

    spec = pl.BlockSpec((1024, TH), lambda i: (i, 0))
    return pl.pallas_call(
        body,
        grid=(NODES_PAD // 1024,),
        in_specs=[spec, spec, spec, spec, spec, spec,
                  pl.BlockSpec((1024, 1), lambda i: (i, 0)),
                  pl.BlockSpec((1, TH), lambda i: (0, 0)),
                  pl.BlockSpec((1, TH), lambda i: (0, 0))],
        out_specs=[spec, spec],
        out_shape=[
            jax.ShapeDtypeStruct((NODES_PAD, TH), jnp.float32),
            jax.ShapeDtypeStruct((NODES_PAD, TH), jnp.float32),
        ],
    )(pK0, pK1, hsK, pV0, pV1, hsV, dinv, bk, bv)


def _heads(t):
    return [t[:, h * NH:(h + 1) * NH] for h in range(HEADS)]


def _attend_tail(o, Wo, bo):
    return o + jnp.maximum(_doth(o, Wo) + bo, 0.0)


def _tc_attn(starts, Kd, Vd, Qs, q3s, w):
    def body(starts_ref, kd_ref, vd_ref, qs_ref, q3_ref,
             wo1, bo1, wq, bq, wk, bk, wv, bv, wso, bso,
             wk2, bk2, wv2, bv2, wo2, bo2,
             l2w, l2b, f0w, f0b, fw, fb, o_ref):
        g = pl.program_id(0)
        base = g * GPB
        s_g = [starts_ref[base + i] for i in range(GPB + 1)]
        start0 = s_g[0]
        nch = (s_g[GPB] - start0 + CL - 1) // CL

        rg = lax.broadcasted_iota(jnp.int32, (NQ, 1), 0) // SEEDS1
        rs = jnp.zeros((NQ, 1), jnp.int32)
        re = jnp.zeros((NQ, 1), jnp.int32)
        for gg in range(GPB):
            rs = jnp.where(rg == gg, s_g[gg], rs)
            re = jnp.where(rg == gg, s_g[gg + 1], re)

        Qsv = qs_ref[...]
        Qh = _heads(Qsv)
        Qh75 = [q[0:SEEDS1, :] for q in Qh]

        def chunk(j, carry):
            off = start0 + j * CL
            kc = kd_ref[pl.ds(off, CL), :]
            vc = vd_ref[pl.ds(off, CL), :]
            pos = off + lax.broadcasted_iota(jnp.int32, (1, CL), 1)
            valid = (pos >= rs) & (pos < re)
            kch = _heads(kc)
            vch = _heads(vc)
            new = []
            for h in range(HEADS):
                m, l, acc = carry[h]
                sc = _dotg(Qh75[h], kch[h], ((1,), (1,))) * SCALE
                sc = jnp.concatenate([sc] * GPB, axis=0)
                sc = jnp.where(valid, sc, NEG)
                mnew = jnp.maximum(m, jnp.max(sc, axis=1, keepdims=True))
                alpha = jnp.exp(m - mnew)
                p = jnp.where(valid, jnp.exp(sc - mnew), 0.0)
                lnew = l * alpha + jnp.sum(p, axis=1, keepdims=True)
                accnew = acc * alpha + _dotg(p, vch[h], ((1,), (0,)))
                new.append((mnew, lnew, accnew))
            return tuple(new)

        init = tuple((jnp.full((NQ, 1), NEG, jnp.float32),
                      jnp.zeros((NQ, 1), jnp.float32),
                      jnp.zeros((NQ, NH), jnp.float32)) for _ in range(HEADS))
        fin = lax.fori_loop(0, nch, chunk, init)
        outs = []
        for h in range(HEADS):
            m, l, acc = fin[h]
            outs.append(Qh[h] + acc / jnp.maximum(l, 1e-20))
        o1 = jnp.concatenate(outs, axis=1)
        h1 = _attend_tail(o1, wo1[...], bo1[...])

        qs_ = _doth(h1, wq[...]) + bq[...]
        ks_ = _doth(h1, wk[...]) + bk[...]
        vs_ = _doth(h1, wv[...]) + bv[...]
        cg = lax.broadcasted_iota(jnp.int32, (1, NQ), 1) // SEEDS1
        bm = rg == cg
        qh2, kh2, vh2 = _heads(qs_), _heads(ks_), _heads(vs_)
        outs2 = []
        for h in range(HEADS):
            sc = _dotg(qh2[h], kh2[h], ((1,), (1,))) * SCALE
            sc = jnp.where(bm, sc, NEG)
            m = jnp.max(sc, axis=1, keepdims=True)
            p = jnp.exp(sc - m)
            p = jnp.where(bm, p, 0.0)
            A = p / jnp.sum(p, axis=1, keepdims=True)
            outs2.append(qh2[h] + _dotg(A, vh2[h], ((1,), (0,))))
        o2 = jnp.concatenate(outs2, axis=1)
        h2 = _attend_tail(o2, wso[...], bso[...])

        q3v = q3_ref[...]
        k3 = _doth(h2, wk2[...]) + bk2[...]
        v3 = _doth(h2, wv2[...]) + bv2[...]
        qg = lax.broadcasted_iota(jnp.int32, (GPB, 1), 0)
        qm = qg == cg
        q3h, k3h, v3h = _heads(q3v), _heads(k3), _heads(v3)
        outs3 = []
        for h in range(HEADS):
            sc = _dotg(q3h[h], k3h[h], ((1,), (1,))) * SCALE
            sc = jnp.where(qm, sc, NEG)
            m = jnp.max(sc, axis=1, keepdims=True)
            p = jnp.exp(sc - m)
            p = jnp.where(qm, p, 0.0)
            A = p / jnp.sum(p, axis=1, keepdims=True)
            outs3.append(q3h[h] + _dotg(A, v3h[h], ((1,), (0,))))
        o3 = jnp.concatenate(outs3, axis=1)
        h3 = _attend_tail(o3, wo2[...], bo2[...])

        g1 = _doth(h3, l2w[...]) + l2b[...]
        g2 = jnp.maximum(_doth(g1, f0w[...]) + f0b[...], 0.0)
        o_ref[...] = _doth(g2, fw[...]) + fb[...]

    full = lambda a: pl.BlockSpec(a.shape, lambda g, s: (0,) * a.ndim)
    win = [w['pma1_fco_W'], w['pma1_fco_b'],
           w['sab_fcq_W'], w['sab_fcq_b'], w['sab_fck_W'], w['sab_fck_b'],
           w['sab_fcv_W'], w['sab_fcv_b'], w['sab_fco_W'], w['sab_fco_b'],
           w['pma2_fck_W'], w['pma2_fck_b'], w['pma2_fcv_W'], w['pma2_fcv_b'],
           w['pma2_fco_W'], w['pma2_fco_b'],
           w['lin2_W'], w['lin2_b'], w['fc0_W'], w['fc0_b'],
           w['final_W'], w['final_b']]
    grid_spec = pltpu.PrefetchScalarGridSpec(
        num_scalar_prefetch=1,
        grid=(NGRAPHS // GPB,),
        in_specs=[full(Kd), full(Vd), full(Qs), full(q3s)] +
                 [full(a) for a in win],
        out_specs=pl.BlockSpec((GPB, 1), lambda g, s: (g, 0)),
    )
    return pl.pallas_call(
        body,
        grid_spec=grid_spec,
        out_shape=jax.ShapeDtypeStruct((NGRAPHS, 1), jnp.float32),
    )(starts, Kd, Vd, Qs, q3s, *win)


def kernel(x, edge_index, edge_attr, batch, params):
    p = params
    f32 = jnp.float32
    i32 = jnp.int32
    src = edge_index[0].astype(i32)
    dst = edge_index[1].astype(i32)

    epad = EPAD - EDGES
    src_idx = jnp.concatenate([src, jnp.zeros((epad,), i32)]).reshape(NW, ECHUNKS, CL)
    dst_idx = jnp.concatenate([dst, jnp.full((epad,), DUMMY, i32)]).reshape(NW, ECHUNKS, CL)
    x_pad = jnp.pad(x, ((0, NODES_PAD - NODES), (0, 0)))
    ea_pad = jnp.pad(edge_attr, ((0, epad), (0, 2)))
    zeros16 = jnp.zeros((NODES_PAD, NH), f32)
    zeros128 = jnp.zeros((NODES_PAD, TH), f32)
    ones_msg = jnp.ones((EPAD, NH), f32)

    r1 = lambda a: a.reshape(1, -1)
    W1p = jnp.pad(p['We1'], ((0, 2), (0, 0)))

    node = _node_proj(x_pad, p['proj_W'], r1(p['proj_b']))
    theta = _edge_mlp(ea_pad, W1p, r1(p['be1']), p['We2'], r1(p['be2']))
    deg_parts = _sc_scatter16(ones_msg, dst_idx, zeros16)

    WihT = p['W_ih'].T
    WhhT = p['W_hh'].T
    h = node
    for _ in range(3):
        xs = _sc_gather16(h, src_idx)
        msg = _tc_msg(xs, theta)
        parts = _sc_scatter16(msg, dst_idx, zeros16)
        h = _tc_update(parts[0], parts[1], h, p['Wroot'], r1(p['broot']),
                       WihT, r1(p['b_ih']), WhhT, r1(p['b_hh']))

    hsK, hsV, dinv = _tc_pre_gcn(h, deg_parts[0], deg_parts[1],
                                 p['lin1_W'], r1(p['lin1_b']),
                                 p['pma1_k_W'], p['pma1_v_W'])
    pK = _sc_gcn128(hsK, src_idx, dst_idx, zeros128)
    pV = _sc_gcn128(hsV, src_idx, dst_idx, zeros128)
    Kd, Vd = _tc_post_gcn(pK[0], pK[1], hsK, pV[0], pV[1], hsV, dinv,
                          r1(p['pma1_k_b']), r1(p['pma1_v_b']))

    Qp = p['S1'][0] @ p['pma1_fcq_W'] + p['pma1_fcq_b']
    Qs = jnp.tile(Qp, (GPB, 1))
    q3 = p['S2'][0] @ p['pma2_fcq_W'] + p['pma2_fcq_b']
    q3s = jnp.tile(q3, (GPB, 1))

    starts = jnp.searchsorted(batch.astype(i32), jnp.arange(NGRAPHS + 1, dtype=i32)).astype(i32)

    wdict = {k: (p[k] if k.endswith('_W') else r1(p[k]))
             for k in ['pma1_fco_W', 'pma1_fco_b',
                       'sab_fcq_W', 'sab_fcq_b', 'sab_fck_W', 'sab_fck_b',
                       'sab_fcv_W', 'sab_fcv_b', 'sab_fco_W', 'sab_fco_b',
                       'pma2_fck_W', 'pma2_fck_b', 'pma2_fcv_W', 'pma2_fcv_b',
                       'pma2_fco_W', 'pma2_fco_b',
                       'lin2_W', 'lin2_b', 'fc0_W', 'fc0_b',
                       'final_W', 'final_b']}
    return _tc_attn(starts, Kd, Vd, Qs, q3s, wdict)
